# baseline (speedup 1.0000x reference)
import functools

import jax
import jax.numpy as jnp
from jax import lax
from jax.experimental import pallas as pl
from jax.experimental.pallas import tpu as pltpu
from jax.experimental.pallas import tpu_sc as plsc

_NC = 2
_NS = 16
_NW = _NC * _NS
_SUB = 128
_K = 12
_G = 6
_MEGA = 40


def _sc_gather(idx2d, W):
    n_chunks, sub = idx2d.shape
    D = W.shape[1]
    B = n_chunks * sub
    nsub = n_chunks // _NW
    nblk = nsub // _MEGA

    mesh = plsc.VectorSubcoreMesh(core_axis_name="c", subcore_axis_name="s")

    @functools.partial(
        pl.kernel,
        out_type=jax.ShapeDtypeStruct((B, D), jnp.float32),
        mesh=mesh,
        scratch_types=[
            pltpu.VMEM((2, _MEGA, _SUB), jnp.int32),
            pltpu.VMEM((_K, _SUB, D), jnp.float32),
            pltpu.SemaphoreType.DMA,
            pltpu.SemaphoreType.DMA,
            pltpu.SemaphoreType.DMA,
        ],
        compiler_params=pltpu.CompilerParams(use_tc_tiling_on_sc=False),
    )
    def k(idx_hbm, w_hbm, out_hbm, idx_v, rows_v, isem, gsem, ssem):
        wid = lax.axis_index("s") * _NC + lax.axis_index("c")
        base_sub = wid * nsub

        def wait_idx():
            pltpu.make_async_copy(
                idx_hbm.at[pl.ds(base_sub, _MEGA)], idx_v.at[0], isem
            ).wait()

        def wait_gather():
            pltpu.make_async_copy(
                w_hbm.at[idx_v.at[0, 0]], rows_v.at[0], gsem
            ).wait()

        def wait_store():
            pltpu.make_async_copy(
                rows_v.at[0], out_hbm.at[pl.ds(0, _SUB)], ssem
            ).wait()

        def fire_store(j, slot):
            pltpu.async_copy(
                rows_v.at[slot],
                out_hbm.at[pl.ds((base_sub + j) * _SUB, _SUB)],
                ssem,
            )

        pltpu.async_copy(idx_hbm.at[pl.ds(base_sub, _MEGA)], idx_v.at[0], isem)

        def body(i, carry):
            s = i % _K
            blk = i // _MEGA
            q = blk % 2
            r = i % _MEGA

            @pl.when(r == 0)
            def _():
                wait_idx()

            @pl.when(jnp.logical_and(r == _G, blk + 1 < nblk))
            def _():
                pltpu.async_copy(
                    idx_hbm.at[pl.ds(base_sub + (blk + 1) * _MEGA, _MEGA)],
                    idx_v.at[1 - q],
                    isem,
                )

            @pl.when(i >= _K)
            def _():
                wait_store()

            pltpu.async_copy(w_hbm.at[idx_v.at[q, r]], rows_v.at[s], gsem)

            @pl.when(i >= _G)
            def _():
                wait_gather()
                fire_store(i - _G, (i - _G) % _K)

            return carry

        lax.fori_loop(0, nsub, body, 0)

        for t in range(_G):
            j = nsub - _G + t
            wait_gather()
            fire_store(j, j % _K)
        for _t in range(_K):
            wait_store()

    return k(idx2d, W)


_BU = 4096


def _tc_transpose(tmp, T, S, D):
    x = tmp.reshape(T, S // 2, 2 * D)

    def body(x_ref, o_ref):
        xb = x_ref[0]
        o_ref[0, :, :_BU] = jnp.transpose(xb[:, :D], (1, 0))
        o_ref[0, :, _BU:] = jnp.transpose(xb[:, D:], (1, 0))

    return pl.pallas_call(
        body,
        grid=(T, S // (2 * _BU)),
        in_specs=[pl.BlockSpec((1, _BU, 2 * D), lambda t, b: (t, b, 0))],
        out_specs=pl.BlockSpec((1, D, 2 * _BU), lambda t, b: (t, 0, b)),
        out_shape=jax.ShapeDtypeStruct((T, D, S), jnp.float32),
    )(x)


def kernel(global_frame_idx, W):
    S, T = global_frame_idx.shape
    D = W.shape[1]
    B = S * T
    nb = S // (2 * _BU)
    idxp = (
        global_frame_idx.T.astype(jnp.int32)
        .reshape(T, nb, 2, _BU)
        .swapaxes(2, 3)
        .reshape(B // _SUB, _SUB)
    )
    tmp = _sc_gather(idxp, W)
    out_phys = _tc_transpose(tmp, T, S, D)
    return jnp.transpose(out_phys, (2, 0, 1))


# speedup vs baseline: 1.2929x; 1.2929x over previous
import functools

import jax
import jax.numpy as jnp
from jax import lax
from jax.experimental import pallas as pl
from jax.experimental.pallas import tpu as pltpu
from jax.experimental.pallas import tpu_sc as plsc

_NC = 2
_NS = 16
_NW = _NC * _NS
_SUB = 128
_K = 12
_G = 6
_MEGA = 40


def _sc_gather(idx2d, W):
    n_chunks, sub = idx2d.shape
    D = W.shape[1]
    B = n_chunks * sub
    nsub = n_chunks // _NW
    nblk = nsub // _MEGA

    mesh = plsc.VectorSubcoreMesh(core_axis_name="c", subcore_axis_name="s")

    @functools.partial(
        pl.kernel,
        out_type=jax.ShapeDtypeStruct((B, D), jnp.float32),
        mesh=mesh,
        scratch_types=[
            pltpu.VMEM((2, _MEGA, _SUB), jnp.int32),
            pltpu.VMEM((_K, _SUB, D), jnp.float32),
            pltpu.SemaphoreType.DMA,
            pltpu.SemaphoreType.DMA,
            pltpu.SemaphoreType.DMA,
        ],
        compiler_params=pltpu.CompilerParams(use_tc_tiling_on_sc=False),
    )
    def k(idx_hbm, w_hbm, out_hbm, idx_v, rows_v, isem, gsem, ssem):
        wid = lax.axis_index("s") * _NC + lax.axis_index("c")
        base_sub = wid * nsub

        def wait_idx():
            pltpu.make_async_copy(
                idx_hbm.at[pl.ds(base_sub, _MEGA)], idx_v.at[0], isem
            ).wait()

        def wait_gather():
            pltpu.make_async_copy(
                w_hbm.at[idx_v.at[0, 0]], rows_v.at[0], gsem
            ).wait()

        def wait_store():
            pltpu.make_async_copy(
                rows_v.at[0], out_hbm.at[pl.ds(0, _SUB)], ssem
            ).wait()

        def fire_store(j, slot):
            pltpu.async_copy(
                rows_v.at[slot],
                out_hbm.at[pl.ds((base_sub + j) * _SUB, _SUB)],
                ssem,
            )

        pltpu.async_copy(idx_hbm.at[pl.ds(base_sub, _MEGA)], idx_v.at[0], isem)

        def body(i, carry):
            s = i % _K
            blk = i // _MEGA
            q = blk % 2
            r = i % _MEGA

            @pl.when(r == 0)
            def _():
                wait_idx()

            @pl.when(jnp.logical_and(r == _G, blk + 1 < nblk))
            def _():
                pltpu.async_copy(
                    idx_hbm.at[pl.ds(base_sub + (blk + 1) * _MEGA, _MEGA)],
                    idx_v.at[1 - q],
                    isem,
                )

            @pl.when(i >= _K)
            def _():
                wait_store()

            pltpu.async_copy(w_hbm.at[idx_v.at[q, r]], rows_v.at[s], gsem)

            @pl.when(i >= _G)
            def _():
                wait_gather()
                fire_store(i - _G, (i - _G) % _K)

            return carry

        lax.fori_loop(0, nsub, body, 0)

        for t in range(_G):
            j = nsub - _G + t
            wait_gather()
            fire_store(j, j % _K)
        for _t in range(_K):
            wait_store()

    return k(idx2d, W)


def kernel(global_frame_idx, W):
    S, T = global_frame_idx.shape
    D = W.shape[1]
    B = S * T
    idx2d = global_frame_idx.astype(jnp.int32).reshape(B // _SUB, _SUB)
    out = _sc_gather(idx2d, W)
    return out.reshape(S, T, D)
